# baseline (speedup 1.0000x reference)
import functools
import math

import numpy as np
import jax
import jax.numpy as jnp
from jax import lax
from jax.experimental import pallas as pl
from jax.experimental.pallas import tpu as pltpu
from jax.experimental.pallas import tpu_sc as plsc

EPS = 1e-5
_INF = float(np.inf)


def _fps_body(p8_ref, sel_ref, ps_ref):
    m = sel_ref.shape[0]
    rows = 8
    W = p8_ref.shape[1]
    px8 = p8_ref[0:rows, :]
    py8 = p8_ref[rows:2 * rows, :]
    pz8 = p8_ref[2 * rows:3 * rows, :]
    flat_ids = (lax.broadcasted_iota(jnp.int32, (rows, W), 0) * W
                + lax.broadcasted_iota(jnp.int32, (rows, W), 1))
    zf = jnp.zeros((rows, W), jnp.float32)

    def coords_at(mask):
        fx = jnp.sum(jnp.where(mask, px8, zf))
        fy = jnp.sum(jnp.where(mask, py8, zf))
        fz = jnp.sum(jnp.where(mask, pz8, zf))
        return fx, fy, fz

    x0, y0, z0 = coords_at(flat_ids == 0)
    d0 = ((px8 - x0) ** 2 + (py8 - y0) ** 2) + (pz8 - z0) ** 2
    sel_ref[0] = 0
    ps_ref[0, 0] = x0
    ps_ref[0, 1] = y0
    ps_ref[0, 2] = z0

    def body(i, d):
        gm = jnp.max(d)
        nxt = jnp.min(jnp.where(d == gm, flat_ids, rows * W))
        px, py, pz = coords_at(flat_ids == nxt)
        sel_ref[i] = nxt.astype(jnp.int32)
        ps_ref[i, 0] = px
        ps_ref[i, 1] = py
        ps_ref[i, 2] = pz
        dn = ((px8 - px) ** 2 + (py8 - py) ** 2) + (pz8 - pz) ** 2
        return jnp.minimum(d, dn)

    lax.fori_loop(1, m, body, d0)


def _fps(pos, m, interpret=False):
    n = pos.shape[0]
    rows = 8
    W = n // rows
    p8 = pos.T.reshape(3 * rows, W)
    return pl.pallas_call(
        _fps_body,
        out_shape=[jax.ShapeDtypeStruct((m,), jnp.int32),
                   jax.ShapeDtypeStruct((m, 3), jnp.float32)],
        in_specs=[pl.BlockSpec(memory_space=pltpu.VMEM)],
        out_specs=[pl.BlockSpec(memory_space=pltpu.SMEM),
                   pl.BlockSpec(memory_space=pltpu.SMEM)],
        interpret=interpret,
    )(p8)


def _knn48_body(q_ref, p_ref, s0a_ref, s1a_ref, s0b_ref, s1b_ref,
                o0a_ref, o1a_ref, o0b_ref, o1b_ref, d_ref, nbr_ref):
    R = q_ref.shape[0]
    N = p_ref.shape[1]
    qx = q_ref[:, 0:1]
    qy = q_ref[:, 1:2]
    qz = q_ref[:, 2:3]
    d_ref[:, :] = ((qx - p_ref[0:1, :]) ** 2 + (qy - p_ref[1:2, :]) ** 2) \
        + (qz - p_ref[2:3, :]) ** 2
    colids = lax.broadcasted_iota(jnp.int32, (R, N), 1)
    mv = jnp.min(d_ref[:, :], axis=1, keepdims=True)
    for k in range(48):
        d = d_ref[:, :]
        idx = jnp.min(jnp.where(d == mv, colids, N), axis=1, keepdims=True)
        nbr_ref[:, k:k + 1] = idx
        dm = jnp.where(colids == idx, _INF, d)
        d_ref[:, :] = dm
        mv = jnp.min(dm, axis=1, keepdims=True)

    def sample(sel_ref, out_ref, kmax):
        K = out_ref.shape[1]
        sel = sel_ref[:, :]
        acc = jnp.zeros((R, K), jnp.int32)
        for j in range(kmax):
            acc = jnp.where(sel == j, nbr_ref[:, j:j + 1], acc)
        out_ref[:, :] = acc

    sample(s0a_ref, o0a_ref, 16)
    sample(s1a_ref, o1a_ref, 48)
    sample(s0b_ref, o0b_ref, 16)
    sample(s1b_ref, o1b_ref, 48)


def _knn48_sampled(pos_q, pos_p, sels, block_r=256, interpret=False):
    m = pos_q.shape[0]
    n = pos_p.shape[0]
    R = min(block_r, m)
    sspec = lambda K: pl.BlockSpec((R, K), lambda i: (i, 0))
    return pl.pallas_call(
        _knn48_body,
        grid=(m // R,),
        in_specs=[pl.BlockSpec((R, 3), lambda i: (i, 0)),
                  pl.BlockSpec((3, n), lambda i: (0, 0)),
                  sspec(8), sspec(24), sspec(8), sspec(24)],
        out_specs=[sspec(8), sspec(24), sspec(8), sspec(24)],
        out_shape=[jax.ShapeDtypeStruct((m, 8), jnp.int32),
                   jax.ShapeDtypeStruct((m, 24), jnp.int32),
                   jax.ShapeDtypeStruct((m, 8), jnp.int32),
                   jax.ShapeDtypeStruct((m, 24), jnp.int32)],
        scratch_shapes=[pltpu.VMEM((R, n), jnp.float32),
                        pltpu.VMEM((R, 48), jnp.int32)],
        interpret=interpret,
    )(pos_q, pos_p.T, *sels)


def _knn3_body(q_ref, p_ref, nbr_ref, w_ref, d_ref):
    R = q_ref.shape[0]
    N = p_ref.shape[1]
    qx = q_ref[:, 0:1]
    qy = q_ref[:, 1:2]
    qz = q_ref[:, 2:3]
    d_ref[:, :] = ((qx - p_ref[0:1, :]) ** 2 + (qy - p_ref[1:2, :]) ** 2) \
        + (qz - p_ref[2:3, :]) ** 2
    colids = lax.broadcasted_iota(jnp.int32, (R, N), 1)
    vals = []
    mv = jnp.min(d_ref[:, :], axis=1, keepdims=True)
    for k in range(3):
        d = d_ref[:, :]
        idx = jnp.min(jnp.where(d == mv, colids, N), axis=1, keepdims=True)
        nbr_ref[:, k:k + 1] = idx
        vals.append(mv)
        dm = jnp.where(colids == idx, _INF, d)
        d_ref[:, :] = dm
        mv = jnp.min(dm, axis=1, keepdims=True)
    d3 = jnp.concatenate(vals, axis=1)
    w = 1.0 / jnp.maximum(d3, 1e-16)
    w_ref[:, :] = w / jnp.sum(w, axis=1, keepdims=True)


def _knn3_weights(pos_q, pos_p, block_r=512, interpret=False):
    m = pos_q.shape[0]
    n = pos_p.shape[0]
    R = min(block_r, m)
    return pl.pallas_call(
        _knn3_body,
        grid=(m // R,),
        in_specs=[pl.BlockSpec((R, 3), lambda i: (i, 0)),
                  pl.BlockSpec((3, n), lambda i: (0, 0))],
        out_specs=[pl.BlockSpec((R, 3), lambda i: (i, 0)),
                   pl.BlockSpec((R, 3), lambda i: (i, 0))],
        out_shape=[jax.ShapeDtypeStruct((m, 3), jnp.int32),
                   jax.ShapeDtypeStruct((m, 3), jnp.float32)],
        scratch_shapes=[pltpu.VMEM((R, n), jnp.float32)],
        interpret=interpret,
    )(pos_q, pos_p.T)


def _sc_gather(table, idx):
    V, D = table.shape
    B = idx.shape[0]
    info = plsc.get_sparse_core_info()
    NC, NS = info.num_cores, info.num_subcores
    NW = NC * NS
    b_per_w = B // NW
    assert b_per_w * NW == B and b_per_w % 8 == 0 and D % 128 == 0, (B, D)
    c = min(b_per_w, 128)
    while b_per_w % c:
        c -= 8
    nch = b_per_w // c
    idx2 = idx.reshape(NW, nch, c)
    mesh = plsc.VectorSubcoreMesh(core_axis_name="c", subcore_axis_name="s",
                                  num_cores=NC)

    @functools.partial(
        pl.kernel, mesh=mesh,
        out_type=jax.ShapeDtypeStruct((B, D), jnp.float32),
        scratch_types=[pltpu.VMEM((nch, c), jnp.int32),
                       pltpu.VMEM((c, D), jnp.float32),
                       pltpu.VMEM((c, D), jnp.float32),
                       pltpu.SemaphoreType.DMA],
    )
    def gk(table_hbm, idx_hbm, out_hbm, idx_v, buf0, buf1, sem):
        wid = lax.axis_index("s") * NC + lax.axis_index("c")
        base = wid * b_per_w
        pltpu.sync_copy(idx_hbm.at[wid], idx_v)
        bufs = (buf0, buf1)
        cps = []
        for j in range(nch):
            cps.append(pltpu.async_copy(table_hbm.at[idx_v.at[j]],
                                        bufs[j % 2], sem))
            if j > 0:
                cps[j - 1].wait()
                pltpu.sync_copy(bufs[(j - 1) % 2],
                                out_hbm.at[pl.ds(base + (j - 1) * c, c)])
        cps[nch - 1].wait()
        pltpu.sync_copy(bufs[(nch - 1) % 2],
                        out_hbm.at[pl.ds(base + (nch - 1) * c, c)])

    return gk(table, idx2)


def _resblock_compute(feat, w):
    (W1, b1, s1, t1, W2, b2, s2, t2, Wd, bd, sd, td) = w
    h1 = (jnp.dot(feat, W1, preferred_element_type=jnp.float32) + b1) * s1 + t1
    h1 = jnp.maximum(h1, 0.0)
    h2 = (jnp.dot(h1, W2, preferred_element_type=jnp.float32) + b2) * s2 + t2
    dn = (jnp.dot(feat, Wd, preferred_element_type=jnp.float32) + bd) * sd + td
    return jnp.maximum(h2 + dn, 0.0)


def _prep_block(block, cp):
    layers, down = block
    (W1, b1, g1, be1), (W2, b2, g2, be2) = layers
    Wd, bd, gd, bed = down
    inv = 1.0 / jnp.sqrt(jnp.float32(1.0 + EPS))

    def padw(W):
        return jnp.pad(W, ((0, cp - W.shape[0]), (0, 0)))

    def row(v):
        return v.reshape(1, -1)

    return (padw(W1), row(b1), row(g1 * inv), row(be1),
            W2, row(b2), row(g2 * inv), row(be2),
            padw(Wd), row(bd), row(gd * inv), row(bed))


_WSPECS = None


def _weight_specs():
    full = pl.BlockSpec(memory_space=pltpu.VMEM)
    return [full] * 12


def _sa_branch_body(C, K, g_ref, q_ref, *refs):
    w = [r[:, :] for r in refs[:-1]]
    o_ref = refs[-1]
    R = q_ref.shape[0]
    Cp = g_ref.shape[1]
    g = g_ref[:, :]
    q = q_ref[:, :]
    qfull = jnp.concatenate(
        [jnp.zeros((R, C), jnp.float32), q,
         jnp.zeros((R, Cp - C - 3), jnp.float32)], axis=1)
    qexp = jnp.broadcast_to(qfull[:, None, :], (R, K, Cp)).reshape(R * K, Cp)
    feat = g - qexp
    out = _resblock_compute(feat, w)
    Co = out.shape[1]
    o_ref[:, :] = jnp.max(out.reshape(R, K, Co), axis=1)


def _sa_branch(gathered, pos_s, block, C, K, row_offset=0, interpret=False):
    m = pos_s.shape[0]
    Cp = gathered.shape[1]
    w = _prep_block(block, Cp)
    Co = w[8].shape[1]
    R = min(128 if K == 8 else 64, m)
    assert row_offset % (R * K) == 0
    off = row_offset // (R * K)
    return pl.pallas_call(
        functools.partial(_sa_branch_body, C, K),
        grid=(m // R,),
        in_specs=[pl.BlockSpec((R * K, Cp), lambda i: (i + off, 0)),
                  pl.BlockSpec((R, 3), lambda i: (i, 0))] + _weight_specs(),
        out_specs=pl.BlockSpec((R, Co), lambda i: (i, 0)),
        out_shape=jax.ShapeDtypeStruct((m, Co), jnp.float32),
        interpret=interpret,
    )(gathered, pos_s, *w)


def _mid_body(x_ref, q_ref, *refs):
    wa = [r[:, :] for r in refs[:12]]
    wb = [r[:, :] for r in refs[12:24]]
    o_ref = refs[24]
    x = x_ref[:, :]
    feat = jnp.concatenate([x, q_ref[:, :]], axis=1)
    h = _resblock_compute(feat, wa)
    gm = jnp.max(h, axis=0, keepdims=True)
    m = x.shape[0]
    xi = jnp.broadcast_to(gm, (m, gm.shape[1]))
    feat2 = jnp.concatenate([xi, x], axis=1)
    o_ref[:, :] = _resblock_compute(feat2, wb)


def _mid(sa2_x, pos2, block_sa3, block_fp3, interpret=False):
    m, C = sa2_x.shape
    wa = _prep_block(block_sa3, C + 3)
    wb = _prep_block(block_fp3, wa[4].shape[1] + C)
    Co = wb[8].shape[1]
    return pl.pallas_call(
        _mid_body,
        in_specs=[pl.BlockSpec(memory_space=pltpu.VMEM)] * 26,
        out_specs=pl.BlockSpec(memory_space=pltpu.VMEM),
        out_shape=jax.ShapeDtypeStruct((m, Co), jnp.float32),
        interpret=interpret,
    )(sa2_x, pos2, *wa, *wb)


def _fp_body(g0_ref, g1_ref, g2_ref, w_ref, skip_ref, *refs):
    w = [r[:, :] for r in refs[:-1]]
    o_ref = refs[-1]
    wt = w_ref[:, :]
    xi = (g0_ref[:, :] * wt[:, 0:1] + g1_ref[:, :] * wt[:, 1:2]) \
        + g2_ref[:, :] * wt[:, 2:3]
    feat = jnp.concatenate([xi, skip_ref[:, :]], axis=1)
    o_ref[:, :] = _resblock_compute(feat, w)


def _fp(gathered3, wts, skip, block, interpret=False):
    m, Cs = skip.shape
    C = gathered3.shape[1]
    w = _prep_block(block, C + Cs)
    Co = w[8].shape[1]
    R = min(512, m)
    nblk = m // R
    gspec = lambda k: pl.BlockSpec((R, C), lambda i, _k=k: (i + _k * nblk, 0))
    return pl.pallas_call(
        _fp_body,
        grid=(nblk,),
        in_specs=[gspec(0), gspec(1), gspec(2),
                  pl.BlockSpec((R, 3), lambda i: (i, 0)),
                  pl.BlockSpec((R, Cs), lambda i: (i, 0))] + _weight_specs(),
        out_specs=pl.BlockSpec((R, Co), lambda i: (i, 0)),
        out_shape=jax.ShapeDtypeStruct((m, Co), jnp.float32),
        interpret=interpret,
    )(gathered3, gathered3, gathered3, wts, skip, *w)


def _head_body(x_ref, w1_ref, b1_ref, w2_ref, b2_ref, w3_ref, b3_ref, o_ref):
    h = jnp.dot(x_ref[:, :], w1_ref[:, :],
                preferred_element_type=jnp.float32) + b1_ref[:, :]
    h = jnp.maximum(h, 0.0)
    h = jnp.dot(h, w2_ref[:, :],
                preferred_element_type=jnp.float32) + b2_ref[:, :]
    h = jnp.dot(h, w3_ref[:, :],
                preferred_element_type=jnp.float32) + b3_ref[:, :]
    mx = jnp.max(h, axis=1, keepdims=True)
    sh = h - mx
    o_ref[:, :] = sh - jnp.log(jnp.sum(jnp.exp(sh), axis=1, keepdims=True))


def _head(xf, lin1, lin2, lin3, interpret=False):
    m, C = xf.shape
    R = min(1024, m)
    nc = lin3[0].shape[1]
    full = pl.BlockSpec(memory_space=pltpu.VMEM)
    return pl.pallas_call(
        _head_body,
        grid=(m // R,),
        in_specs=[pl.BlockSpec((R, C), lambda i: (i, 0)),
                  full, full, full, full, full, full],
        out_specs=pl.BlockSpec((R, nc), lambda i: (i, 0)),
        out_shape=jax.ShapeDtypeStruct((m, nc), jnp.float32),
        interpret=interpret,
    )(xf, lin1[0], lin1[1].reshape(1, -1), lin2[0], lin2[1].reshape(1, -1),
      lin3[0], lin3[1].reshape(1, -1))


def _pad_cols(a, D):
    return jnp.pad(a, ((0, 0), (0, D - a.shape[1])))


def _pad128(c):
    return ((c + 127) // 128) * 128


def kernel(x, pos, batch, params):
    N = pos.shape[0]
    m1 = int(math.ceil(0.25 * N))
    m2 = int(math.ceil(0.25 * m1))
    key = jax.random.key(42)

    def draw(sub, m):
        return (jax.random.randint(jax.random.fold_in(sub, 0), (m, 8), 0, 16),
                jax.random.randint(jax.random.fold_in(sub, 1), (m, 24), 0, 48))

    s0_sa1, s1_sa1 = draw(jax.random.fold_in(key, 1), m1)
    s0_sa2, s1_sa2 = draw(jax.random.fold_in(key, 2), m2)
    s0_cr1, s1_cr1 = draw(jax.random.fold_in(key, 100), m1)
    s0_cr2, s1_cr2 = draw(jax.random.fold_in(key, 101), m2)

    sel1, pos1 = _fps(pos, m1)
    o0_sa1, o1_sa1, o0_cr1, o1_cr1 = _knn48_sampled(
        pos1, pos, (s0_sa1, s1_sa1, s0_cr1, s1_cr1))
    sel2, pos2 = _fps(pos1, m2)
    o0_sa2, o1_sa2, o0_cr2, o1_cr2 = _knn48_sampled(
        pos2, pos1, (s0_sa2, s1_sa2, s0_cr2, s1_cr2))
    nbr3_21, w21 = _knn3_weights(pos1, pos2)
    nbr3_10, w10 = _knn3_weights(pos, pos1)
    flat21 = nbr3_21.T.reshape(-1)
    flat10 = nbr3_10.T.reshape(-1)

    def run_round(feats, sels_l1, sels_l2, names):
        sa1n, sa2n, sa3n, fp3n, fp2n, fp1n = names
        C1 = feats.shape[1]
        n1 = sels_l1[1].size
        T1 = _pad_cols(jnp.concatenate([feats, pos], axis=1),
                       _pad128(C1 + 3))
        g = _sc_gather(T1, jnp.concatenate(
            [sels_l1[1].reshape(-1), sels_l1[0].reshape(-1)]))
        b0 = _sa_branch(g, pos1, params[sa1n][0], C1, 8, row_offset=n1)
        b1 = _sa_branch(g, pos1, params[sa1n][1], C1, 24)
        sa1_x = jnp.concatenate([b0, b1], axis=1)

        C2 = sa1_x.shape[1]
        n1 = sels_l2[1].size
        T2 = _pad_cols(jnp.concatenate([sa1_x, pos1], axis=1),
                       _pad128(C2 + 3))
        g = _sc_gather(T2, jnp.concatenate(
            [sels_l2[1].reshape(-1), sels_l2[0].reshape(-1)]))
        b0 = _sa_branch(g, pos2, params[sa2n][0], C2, 8, row_offset=n1)
        b1 = _sa_branch(g, pos2, params[sa2n][1], C2, 24)
        sa2_x = jnp.concatenate([b0, b1], axis=1)

        fp3_x = _mid(sa2_x, pos2, params[sa3n], params[fp3n])
        gi = _sc_gather(fp3_x, flat21)
        fp2_x = _fp(gi, w21, sa1_x, params[fp2n])
        gi2 = _sc_gather(fp2_x, flat10)
        fp1_x = _fp(gi2, w10, feats, params[fp1n])
        return fp1_x

    fp0_x = run_round(x, (o0_sa1, o1_sa1), (o0_sa2, o1_sa2),
                      ("sa1", "sa2", "sa3", "fp3", "fp2", "fp1"))
    cr_x = run_round(fp0_x, (o0_cr1, o1_cr1), (o0_cr2, o1_cr2),
                     ("crsa1", "crsa2", "crsa3", "crfp3", "crfp2", "crfp1"))

    return _head(cr_x, params["lin1"], params["lin2"], params["lin3"])


# speedup vs baseline: 1.0233x; 1.0233x over previous
import functools
import math

import numpy as np
import jax
import jax.numpy as jnp
from jax import lax
from jax.experimental import pallas as pl
from jax.experimental.pallas import tpu as pltpu
from jax.experimental.pallas import tpu_sc as plsc

EPS = 1e-5
_INF = float(np.inf)


def _fps_body(p8_ref, sel_ref, ps_ref):
    m = sel_ref.shape[0]
    rows = 8
    W = p8_ref.shape[1]
    px8 = p8_ref[0:rows, :]
    py8 = p8_ref[rows:2 * rows, :]
    pz8 = p8_ref[2 * rows:3 * rows, :]
    flat_ids = (lax.broadcasted_iota(jnp.int32, (rows, W), 0) * W
                + lax.broadcasted_iota(jnp.int32, (rows, W), 1))
    zf = jnp.zeros((rows, W), jnp.float32)

    def coords_at(mask):
        fx = jnp.sum(jnp.where(mask, px8, zf))
        fy = jnp.sum(jnp.where(mask, py8, zf))
        fz = jnp.sum(jnp.where(mask, pz8, zf))
        return fx, fy, fz

    x0, y0, z0 = coords_at(flat_ids == 0)
    d0 = ((px8 - x0) ** 2 + (py8 - y0) ** 2) + (pz8 - z0) ** 2
    sel_ref[0] = 0
    ps_ref[0, 0] = x0
    ps_ref[0, 1] = y0
    ps_ref[0, 2] = z0

    def body(i, d):
        gm = jnp.max(d)
        nxt = jnp.min(jnp.where(d == gm, flat_ids, rows * W))
        px, py, pz = coords_at(flat_ids == nxt)
        sel_ref[i] = nxt.astype(jnp.int32)
        ps_ref[i, 0] = px
        ps_ref[i, 1] = py
        ps_ref[i, 2] = pz
        dn = ((px8 - px) ** 2 + (py8 - py) ** 2) + (pz8 - pz) ** 2
        return jnp.minimum(d, dn)

    lax.fori_loop(1, m, body, d0)


def _fps(pos, m, interpret=False):
    n = pos.shape[0]
    rows = 8
    W = n // rows
    p8 = pos.T.reshape(3 * rows, W)
    return pl.pallas_call(
        _fps_body,
        out_shape=[jax.ShapeDtypeStruct((m,), jnp.int32),
                   jax.ShapeDtypeStruct((m, 3), jnp.float32)],
        in_specs=[pl.BlockSpec(memory_space=pltpu.VMEM)],
        out_specs=[pl.BlockSpec(memory_space=pltpu.SMEM),
                   pl.BlockSpec(memory_space=pltpu.SMEM)],
        interpret=interpret,
    )(p8)


def _knn48_body(q_ref, p_ref, s0a_ref, s1a_ref, s0b_ref, s1b_ref,
                o0a_ref, o1a_ref, o0b_ref, o1b_ref, d_ref, nbr_ref):
    R = q_ref.shape[0]
    N = p_ref.shape[1]
    qx = q_ref[:, 0:1]
    qy = q_ref[:, 1:2]
    qz = q_ref[:, 2:3]
    d_ref[:, :] = ((qx - p_ref[0:1, :]) ** 2 + (qy - p_ref[1:2, :]) ** 2) \
        + (qz - p_ref[2:3, :]) ** 2
    colids = lax.broadcasted_iota(jnp.int32, (R, N), 1)
    mv = jnp.min(d_ref[:, :], axis=1, keepdims=True)
    for k in range(48):
        d = d_ref[:, :]
        idx = jnp.min(jnp.where(d == mv, colids, N), axis=1, keepdims=True)
        nbr_ref[:, k:k + 1] = idx
        dm = jnp.where(colids == idx, _INF, d)
        d_ref[:, :] = dm
        mv = jnp.min(dm, axis=1, keepdims=True)

    def sample(sel_ref, out_ref, kmax):
        K = out_ref.shape[1]
        sel = sel_ref[:, :]
        acc = jnp.zeros((R, K), jnp.int32)
        for j in range(kmax):
            acc = jnp.where(sel == j, nbr_ref[:, j:j + 1], acc)
        out_ref[:, :] = acc

    sample(s0a_ref, o0a_ref, 16)
    sample(s1a_ref, o1a_ref, 48)
    sample(s0b_ref, o0b_ref, 16)
    sample(s1b_ref, o1b_ref, 48)


def _knn48_sampled(pos_q, pos_p, sels, block_r=512, interpret=False):
    m = pos_q.shape[0]
    n = pos_p.shape[0]
    R = min(block_r, m)
    sspec = lambda K: pl.BlockSpec((R, K), lambda i: (i, 0))
    return pl.pallas_call(
        _knn48_body,
        grid=(m // R,),
        in_specs=[pl.BlockSpec((R, 3), lambda i: (i, 0)),
                  pl.BlockSpec((3, n), lambda i: (0, 0)),
                  sspec(8), sspec(24), sspec(8), sspec(24)],
        out_specs=[sspec(8), sspec(24), sspec(8), sspec(24)],
        out_shape=[jax.ShapeDtypeStruct((m, 8), jnp.int32),
                   jax.ShapeDtypeStruct((m, 24), jnp.int32),
                   jax.ShapeDtypeStruct((m, 8), jnp.int32),
                   jax.ShapeDtypeStruct((m, 24), jnp.int32)],
        scratch_shapes=[pltpu.VMEM((R, n), jnp.float32),
                        pltpu.VMEM((R, 48), jnp.int32)],
        interpret=interpret,
    )(pos_q, pos_p.T, *sels)


def _knn3_body(q_ref, p_ref, nbr_ref, w_ref, d_ref):
    R = q_ref.shape[0]
    N = p_ref.shape[1]
    qx = q_ref[:, 0:1]
    qy = q_ref[:, 1:2]
    qz = q_ref[:, 2:3]
    d_ref[:, :] = ((qx - p_ref[0:1, :]) ** 2 + (qy - p_ref[1:2, :]) ** 2) \
        + (qz - p_ref[2:3, :]) ** 2
    colids = lax.broadcasted_iota(jnp.int32, (R, N), 1)
    vals = []
    mv = jnp.min(d_ref[:, :], axis=1, keepdims=True)
    for k in range(3):
        d = d_ref[:, :]
        idx = jnp.min(jnp.where(d == mv, colids, N), axis=1, keepdims=True)
        nbr_ref[:, k:k + 1] = idx
        vals.append(mv)
        dm = jnp.where(colids == idx, _INF, d)
        d_ref[:, :] = dm
        mv = jnp.min(dm, axis=1, keepdims=True)
    d3 = jnp.concatenate(vals, axis=1)
    w = 1.0 / jnp.maximum(d3, 1e-16)
    w_ref[:, :] = w / jnp.sum(w, axis=1, keepdims=True)


def _knn3_weights(pos_q, pos_p, block_r=1024, interpret=False):
    m = pos_q.shape[0]
    n = pos_p.shape[0]
    R = min(block_r, m)
    return pl.pallas_call(
        _knn3_body,
        grid=(m // R,),
        in_specs=[pl.BlockSpec((R, 3), lambda i: (i, 0)),
                  pl.BlockSpec((3, n), lambda i: (0, 0))],
        out_specs=[pl.BlockSpec((R, 3), lambda i: (i, 0)),
                   pl.BlockSpec((R, 3), lambda i: (i, 0))],
        out_shape=[jax.ShapeDtypeStruct((m, 3), jnp.int32),
                   jax.ShapeDtypeStruct((m, 3), jnp.float32)],
        scratch_shapes=[pltpu.VMEM((R, n), jnp.float32)],
        interpret=interpret,
    )(pos_q, pos_p.T)


def _sc_gather(table, idx):
    V, D = table.shape
    B = idx.shape[0]
    info = plsc.get_sparse_core_info()
    NC, NS = info.num_cores, info.num_subcores
    NW = NC * NS
    b_per_w = B // NW
    assert b_per_w * NW == B and b_per_w % 8 == 0 and D % 128 == 0, (B, D)
    c = min(b_per_w, 128)
    while b_per_w % c:
        c -= 8
    nch = b_per_w // c
    idx2 = idx.reshape(NW, nch, c)
    mesh = plsc.VectorSubcoreMesh(core_axis_name="c", subcore_axis_name="s",
                                  num_cores=NC)

    @functools.partial(
        pl.kernel, mesh=mesh,
        out_type=jax.ShapeDtypeStruct((B, D), jnp.float32),
        scratch_types=[pltpu.VMEM((nch, c), jnp.int32),
                       pltpu.VMEM((c, D), jnp.float32),
                       pltpu.VMEM((c, D), jnp.float32),
                       pltpu.SemaphoreType.DMA],
    )
    def gk(table_hbm, idx_hbm, out_hbm, idx_v, buf0, buf1, sem):
        wid = lax.axis_index("s") * NC + lax.axis_index("c")
        base = wid * b_per_w
        pltpu.sync_copy(idx_hbm.at[wid], idx_v)
        bufs = (buf0, buf1)
        cps = []
        for j in range(nch):
            cps.append(pltpu.async_copy(table_hbm.at[idx_v.at[j]],
                                        bufs[j % 2], sem))
            if j > 0:
                cps[j - 1].wait()
                pltpu.sync_copy(bufs[(j - 1) % 2],
                                out_hbm.at[pl.ds(base + (j - 1) * c, c)])
        cps[nch - 1].wait()
        pltpu.sync_copy(bufs[(nch - 1) % 2],
                        out_hbm.at[pl.ds(base + (nch - 1) * c, c)])

    return gk(table, idx2)


def _resblock_compute(feat, w):
    (W1, b1, s1, t1, W2, b2, s2, t2, Wd, bd, sd, td) = w
    h1 = (jnp.dot(feat, W1, preferred_element_type=jnp.float32) + b1) * s1 + t1
    h1 = jnp.maximum(h1, 0.0)
    h2 = (jnp.dot(h1, W2, preferred_element_type=jnp.float32) + b2) * s2 + t2
    dn = (jnp.dot(feat, Wd, preferred_element_type=jnp.float32) + bd) * sd + td
    return jnp.maximum(h2 + dn, 0.0)


def _prep_block(block, cp):
    layers, down = block
    (W1, b1, g1, be1), (W2, b2, g2, be2) = layers
    Wd, bd, gd, bed = down
    inv = 1.0 / jnp.sqrt(jnp.float32(1.0 + EPS))

    def padw(W):
        return jnp.pad(W, ((0, cp - W.shape[0]), (0, 0)))

    def row(v):
        return v.reshape(1, -1)

    return (padw(W1), row(b1), row(g1 * inv), row(be1),
            W2, row(b2), row(g2 * inv), row(be2),
            padw(Wd), row(bd), row(gd * inv), row(bed))


_WSPECS = None


def _weight_specs():
    full = pl.BlockSpec(memory_space=pltpu.VMEM)
    return [full] * 12


def _sa_branch_body(C, K, g_ref, q_ref, *refs):
    w = [r[:, :] for r in refs[:-1]]
    o_ref = refs[-1]
    R = q_ref.shape[0]
    Cp = g_ref.shape[1]
    g = g_ref[:, :]
    q = q_ref[:, :]
    qfull = jnp.concatenate(
        [jnp.zeros((R, C), jnp.float32), q,
         jnp.zeros((R, Cp - C - 3), jnp.float32)], axis=1)
    qexp = jnp.broadcast_to(qfull[:, None, :], (R, K, Cp)).reshape(R * K, Cp)
    feat = g - qexp
    out = _resblock_compute(feat, w)
    Co = out.shape[1]
    o_ref[:, :] = jnp.max(out.reshape(R, K, Co), axis=1)


def _sa_branch(gathered, pos_s, block, C, K, row_offset=0, interpret=False):
    m = pos_s.shape[0]
    Cp = gathered.shape[1]
    w = _prep_block(block, Cp)
    Co = w[8].shape[1]
    R = min(128 if K == 8 else 64, m)
    assert row_offset % (R * K) == 0
    off = row_offset // (R * K)
    return pl.pallas_call(
        functools.partial(_sa_branch_body, C, K),
        grid=(m // R,),
        in_specs=[pl.BlockSpec((R * K, Cp), lambda i: (i + off, 0)),
                  pl.BlockSpec((R, 3), lambda i: (i, 0))] + _weight_specs(),
        out_specs=pl.BlockSpec((R, Co), lambda i: (i, 0)),
        out_shape=jax.ShapeDtypeStruct((m, Co), jnp.float32),
        interpret=interpret,
    )(gathered, pos_s, *w)


def _mid_body(x_ref, q_ref, *refs):
    wa = [r[:, :] for r in refs[:12]]
    wb = [r[:, :] for r in refs[12:24]]
    o_ref = refs[24]
    x = x_ref[:, :]
    feat = jnp.concatenate([x, q_ref[:, :]], axis=1)
    h = _resblock_compute(feat, wa)
    gm = jnp.max(h, axis=0, keepdims=True)
    m = x.shape[0]
    xi = jnp.broadcast_to(gm, (m, gm.shape[1]))
    feat2 = jnp.concatenate([xi, x], axis=1)
    o_ref[:, :] = _resblock_compute(feat2, wb)


def _mid(sa2_x, pos2, block_sa3, block_fp3, interpret=False):
    m, C = sa2_x.shape
    wa = _prep_block(block_sa3, C + 3)
    wb = _prep_block(block_fp3, wa[4].shape[1] + C)
    Co = wb[8].shape[1]
    return pl.pallas_call(
        _mid_body,
        in_specs=[pl.BlockSpec(memory_space=pltpu.VMEM)] * 26,
        out_specs=pl.BlockSpec(memory_space=pltpu.VMEM),
        out_shape=jax.ShapeDtypeStruct((m, Co), jnp.float32),
        interpret=interpret,
    )(sa2_x, pos2, *wa, *wb)


def _fp_body(g0_ref, g1_ref, g2_ref, w_ref, skip_ref, *refs):
    w = [r[:, :] for r in refs[:12]]
    o_ref = refs[-1]
    wt = w_ref[:, :]
    xi = (g0_ref[:, :] * wt[:, 0:1] + g1_ref[:, :] * wt[:, 1:2]) \
        + g2_ref[:, :] * wt[:, 2:3]
    feat = jnp.concatenate([xi, skip_ref[:, :]], axis=1)
    out = _resblock_compute(feat, w)
    if len(refs) > 13:
        w1_ref, b1_ref, w2_ref, b2_ref, w3_ref, b3_ref = refs[12:18]
        h = jnp.dot(out, w1_ref[:, :],
                    preferred_element_type=jnp.float32) + b1_ref[:, :]
        h = jnp.maximum(h, 0.0)
        h = jnp.dot(h, w2_ref[:, :],
                    preferred_element_type=jnp.float32) + b2_ref[:, :]
        h = jnp.dot(h, w3_ref[:, :],
                    preferred_element_type=jnp.float32) + b3_ref[:, :]
        mx = jnp.max(h, axis=1, keepdims=True)
        sh = h - mx
        out = sh - jnp.log(jnp.sum(jnp.exp(sh), axis=1, keepdims=True))
    o_ref[:, :] = out


def _fp(gathered3, wts, skip, block, head=None, interpret=False):
    m, Cs = skip.shape
    C = gathered3.shape[1]
    w = _prep_block(block, C + Cs)
    Co = w[8].shape[1]
    R = min(512, m)
    nblk = m // R
    gspec = lambda k: pl.BlockSpec((R, C), lambda i, _k=k: (i + _k * nblk, 0))
    full = pl.BlockSpec(memory_space=pltpu.VMEM)
    extra = []
    if head is not None:
        lin1, lin2, lin3 = head
        extra = [lin1[0], lin1[1].reshape(1, -1), lin2[0],
                 lin2[1].reshape(1, -1), lin3[0], lin3[1].reshape(1, -1)]
        Co = lin3[0].shape[1]
    return pl.pallas_call(
        _fp_body,
        grid=(nblk,),
        in_specs=[gspec(0), gspec(1), gspec(2),
                  pl.BlockSpec((R, 3), lambda i: (i, 0)),
                  pl.BlockSpec((R, Cs), lambda i: (i, 0))] + _weight_specs()
        + [full] * len(extra),
        out_specs=pl.BlockSpec((R, Co), lambda i: (i, 0)),
        out_shape=jax.ShapeDtypeStruct((m, Co), jnp.float32),
        interpret=interpret,
    )(gathered3, gathered3, gathered3, wts, skip, *w, *extra)


def _head_body(x_ref, w1_ref, b1_ref, w2_ref, b2_ref, w3_ref, b3_ref, o_ref):
    h = jnp.dot(x_ref[:, :], w1_ref[:, :],
                preferred_element_type=jnp.float32) + b1_ref[:, :]
    h = jnp.maximum(h, 0.0)
    h = jnp.dot(h, w2_ref[:, :],
                preferred_element_type=jnp.float32) + b2_ref[:, :]
    h = jnp.dot(h, w3_ref[:, :],
                preferred_element_type=jnp.float32) + b3_ref[:, :]
    mx = jnp.max(h, axis=1, keepdims=True)
    sh = h - mx
    o_ref[:, :] = sh - jnp.log(jnp.sum(jnp.exp(sh), axis=1, keepdims=True))


def _head(xf, lin1, lin2, lin3, interpret=False):
    m, C = xf.shape
    R = min(1024, m)
    nc = lin3[0].shape[1]
    full = pl.BlockSpec(memory_space=pltpu.VMEM)
    return pl.pallas_call(
        _head_body,
        grid=(m // R,),
        in_specs=[pl.BlockSpec((R, C), lambda i: (i, 0)),
                  full, full, full, full, full, full],
        out_specs=pl.BlockSpec((R, nc), lambda i: (i, 0)),
        out_shape=jax.ShapeDtypeStruct((m, nc), jnp.float32),
        interpret=interpret,
    )(xf, lin1[0], lin1[1].reshape(1, -1), lin2[0], lin2[1].reshape(1, -1),
      lin3[0], lin3[1].reshape(1, -1))


def _pad_cols(a, D):
    return jnp.pad(a, ((0, 0), (0, D - a.shape[1])))


def _pad128(c):
    return ((c + 127) // 128) * 128


def kernel(x, pos, batch, params):
    N = pos.shape[0]
    m1 = int(math.ceil(0.25 * N))
    m2 = int(math.ceil(0.25 * m1))
    key = jax.random.key(42)

    def draw(sub, m):
        return (jax.random.randint(jax.random.fold_in(sub, 0), (m, 8), 0, 16),
                jax.random.randint(jax.random.fold_in(sub, 1), (m, 24), 0, 48))

    s0_sa1, s1_sa1 = draw(jax.random.fold_in(key, 1), m1)
    s0_sa2, s1_sa2 = draw(jax.random.fold_in(key, 2), m2)
    s0_cr1, s1_cr1 = draw(jax.random.fold_in(key, 100), m1)
    s0_cr2, s1_cr2 = draw(jax.random.fold_in(key, 101), m2)

    sel1, pos1 = _fps(pos, m1)
    o0_sa1, o1_sa1, o0_cr1, o1_cr1 = _knn48_sampled(
        pos1, pos, (s0_sa1, s1_sa1, s0_cr1, s1_cr1))
    sel2, pos2 = _fps(pos1, m2)
    o0_sa2, o1_sa2, o0_cr2, o1_cr2 = _knn48_sampled(
        pos2, pos1, (s0_sa2, s1_sa2, s0_cr2, s1_cr2))
    nbr3_21, w21 = _knn3_weights(pos1, pos2)
    nbr3_10, w10 = _knn3_weights(pos, pos1)
    flat21 = nbr3_21.T.reshape(-1)
    flat10 = nbr3_10.T.reshape(-1)

    def run_round(feats, sels_l1, sels_l2, names, head=None):
        sa1n, sa2n, sa3n, fp3n, fp2n, fp1n = names
        C1 = feats.shape[1]
        n1 = sels_l1[1].size
        T1 = _pad_cols(jnp.concatenate([feats, pos], axis=1),
                       _pad128(C1 + 3))
        g = _sc_gather(T1, jnp.concatenate(
            [sels_l1[1].reshape(-1), sels_l1[0].reshape(-1)]))
        b0 = _sa_branch(g, pos1, params[sa1n][0], C1, 8, row_offset=n1)
        b1 = _sa_branch(g, pos1, params[sa1n][1], C1, 24)
        sa1_x = jnp.concatenate([b0, b1], axis=1)

        C2 = sa1_x.shape[1]
        n1 = sels_l2[1].size
        T2 = _pad_cols(jnp.concatenate([sa1_x, pos1], axis=1),
                       _pad128(C2 + 3))
        g = _sc_gather(T2, jnp.concatenate(
            [sels_l2[1].reshape(-1), sels_l2[0].reshape(-1)]))
        b0 = _sa_branch(g, pos2, params[sa2n][0], C2, 8, row_offset=n1)
        b1 = _sa_branch(g, pos2, params[sa2n][1], C2, 24)
        sa2_x = jnp.concatenate([b0, b1], axis=1)

        fp3_x = _mid(sa2_x, pos2, params[sa3n], params[fp3n])
        gi = _sc_gather(fp3_x, flat21)
        fp2_x = _fp(gi, w21, sa1_x, params[fp2n])
        gi2 = _sc_gather(fp2_x, flat10)
        fp1_x = _fp(gi2, w10, feats, params[fp1n], head=head)
        return fp1_x

    fp0_x = run_round(x, (o0_sa1, o1_sa1), (o0_sa2, o1_sa2),
                      ("sa1", "sa2", "sa3", "fp3", "fp2", "fp1"))
    return run_round(fp0_x, (o0_cr1, o1_cr1), (o0_cr2, o1_cr2),
                     ("crsa1", "crsa2", "crsa3", "crfp3", "crfp2", "crfp1"),
                     head=(params["lin1"], params["lin2"], params["lin3"]))
